# baseline (speedup 1.0000x reference)
import functools

import jax
import jax.numpy as jnp
from jax import lax
from jax.experimental import pallas as pl
from jax.experimental.pallas import tpu as pltpu

KB = 64
LN = 128
TB = 8


def _topk_body(tgt_ref, u_ref, vals_ref, idx_ref, *, rs, nprog_t):
    pid = pl.program_id(0)
    b = pid // nprog_t
    tj = pid % nprog_t
    sub = lax.broadcasted_iota(jnp.int32, (rs, LN), 0)
    lane = lax.broadcasted_iota(jnp.int32, (rs, LN), 1)
    gidx = sub * LN + lane
    lane64 = lax.broadcasted_iota(jnp.int32, (1, KB), 1)
    for r in range(TB):
        t = tj * TB + r
        tgt = tgt_ref[b, t]
        x0 = u_ref[0, r]
        is_t = gidx == tgt
        gold = jnp.sum(jnp.where(is_t, x0, 0.0))
        x = jnp.where(is_t, jnp.inf, x0)

        def body(k, c):
            x, vals, ids = c
            m = jnp.max(x)
            i = jnp.min(jnp.where(x == m, gidx, jnp.int32(2**30)))
            vals = jnp.where(lane64 == k, m, vals)
            ids = jnp.where(lane64 == k, i, ids)
            x = jnp.where(gidx == i, -jnp.inf, x)
            return x, vals, ids

        _, vals, ids = lax.fori_loop(
            0, KB, body,
            (x, jnp.zeros((1, KB), jnp.float32), jnp.zeros((1, KB), jnp.int32)))
        vals = jnp.where(lane64 == 0, gold, vals)
        vals_ref[0, r] = vals.reshape(KB)
        idx_ref[0, r] = ids.reshape(KB)


def _tail_body(mask_smem, vals_ref, g1_ref, g2_ref, mask_ref, out_ref, *, t):
    b = pl.program_id(0)

    @pl.when(b == 0)
    def _():
        out_ref[0, 0] = 0.0

    v = vals_ref[0]
    mx = jnp.max(v, axis=-1, keepdims=True)
    lse = jnp.log(jnp.sum(jnp.exp(v - mx), axis=-1, keepdims=True)) + mx
    node = (v[:, :1] - lse).reshape(1, t)
    msk = mask_ref[0, 0].reshape(1, t)
    node_sum = jnp.sum(node * msk)
    tsum = jnp.sum(msk)

    r64 = lax.broadcasted_iota(jnp.int32, (KB, KB), 0)
    c64 = lax.broadcasted_iota(jnp.int32, (KB, KB), 1)

    def body(i, acc):
        s1 = g1_ref[0, i]
        s2 = g2_ref[0, i + 1]
        m = lax.dot_general(s1, s2, (((1,), (1,)), ((), ())),
                            preferred_element_type=jnp.float32)
        m00 = jnp.sum(jnp.where((r64 == 0) & (c64 == 0), m, 0.0))
        mmx = jnp.max(m)
        else_ = jnp.log(jnp.sum(jnp.exp(m - mmx))) + mmx
        return acc + (m00 - else_) * mask_smem[b, i + 1].astype(jnp.float32)

    edge_sum = lax.fori_loop(0, t - 1, body, jnp.float32(0.0))
    nb = pl.num_programs(0)
    out_ref[0, 0] += -(node_sum + edge_sum) / tsum / nb


def kernel(unaries, masks, targets, E1_weight, E2_weight):
    b, t, v = unaries.shape
    rs = v // LN
    nprog_t = t // TB
    nprog = b * nprog_t
    u4 = unaries.reshape(nprog, TB, rs, LN)

    vals, ids = pl.pallas_call(
        functools.partial(_topk_body, rs=rs, nprog_t=nprog_t),
        grid=(nprog,),
        in_specs=[
            pl.BlockSpec(memory_space=pltpu.SMEM),
            pl.BlockSpec((1, TB, rs, LN), lambda i: (i, 0, 0, 0)),
        ],
        out_specs=[
            pl.BlockSpec((1, TB, KB), lambda i: (i, 0, 0)),
            pl.BlockSpec((1, TB, KB), lambda i: (i, 0, 0)),
        ],
        out_shape=[
            jax.ShapeDtypeStruct((nprog, TB, KB), jnp.float32),
            jax.ShapeDtypeStruct((nprog, TB, KB), jnp.int32),
        ],
    )(targets.astype(jnp.int32), u4)

    vals = vals.reshape(b, t, KB)
    ids = ids.reshape(b, t, KB)

    g1 = jnp.take(E1_weight, ids, axis=0)
    g2 = jnp.take(E2_weight, ids, axis=0)

    nll = pl.pallas_call(
        functools.partial(_tail_body, t=t),
        grid=(b,),
        in_specs=[
            pl.BlockSpec(memory_space=pltpu.SMEM),
            pl.BlockSpec((1, t, KB), lambda i: (i, 0, 0)),
            pl.BlockSpec((1, t, KB, g1.shape[-1]), lambda i: (i, 0, 0, 0)),
            pl.BlockSpec((1, t, KB, g1.shape[-1]), lambda i: (i, 0, 0, 0)),
            pl.BlockSpec((1, 1, t), lambda i: (i, 0, 0)),
        ],
        out_specs=pl.BlockSpec(memory_space=pltpu.SMEM),
        out_shape=jax.ShapeDtypeStruct((1, 1), jnp.float32),
    )(masks.astype(jnp.int32), vals, g1, g2,
      masks.astype(jnp.float32).reshape(b, 1, t))

    return nll[0, 0]
